# Initial kernel scaffold; baseline (speedup 1.0000x reference)
import jax
import jax.numpy as jnp
from jax.experimental import pallas as pl


def kernel(x, edge_index, edge_type, rel_embed, w_loop, w_in, w_out, w_rel, loop_rel, bias, bn_gamma, bn_beta):
    raise NotImplementedError("write your pallas kernel here")


# speedup vs baseline: 3.9805x; 3.9805x over previous
import numpy as np
import jax
import jax.numpy as jnp
from jax import lax
from jax.experimental import pallas as pl
from jax.experimental.pallas import tpu as pltpu
from jax.experimental.pallas import tpu_sc as plsc

N = 10000
D = 128
M = 65
REC = 128
VREC = 256
NP = 10240
RELP = 208
NE = 160000
NEP = 163840
NC = 2
NS = 16
EPT = NEP // NS
BB = 64
NBLK = EPT // BB
RPT = NP // NS

_ang = 2.0 * np.pi * np.outer(np.arange(D), np.arange(M)) / D
_cos = np.cos(_ang)
_sin = np.sin(_ang)
_wj = np.full((M,), 2.0)
_wj[0] = 1.0
_wj[-1] = 1.0

_F2 = np.zeros((D, REC), np.float32)
_F2[:, :64] = _cos[:, :64]
_F2[:, 64] = _cos[:, 64]
_F2[:, 65:] = -_sin[:, 1:64]
_FB = np.zeros((D, REC), np.float32)
_FB[:, :64] = -_sin[:, :64]
_FB[:, 64] = _cos[:, 64]
_FB[:, 65:] = _cos[:, 1:64]
_FD = np.concatenate([_F2, _FB], axis=1)

_Pr = (_wj[:, None] * _cos.T) / D
_Pi = -(_wj[:, None] * _sin.T) / D
_P2 = np.zeros((REC, D), np.float32)
_P2[0] = _Pr[0]
_P2[1:64] = _Pr[1:64]
_P2[64] = _Pr[0] - _Pr[64]
_P2[65:] = _Pi[1:64]

_HI = jax.lax.Precision.HIGHEST


def _dot(a, b):
    return jnp.dot(a, b, preferred_element_type=jnp.float32, precision=_HI)


def _cplx_dense(u, v1, v2):
    ul, uh = u[:, :64], u[:, 64:]
    return jnp.concatenate(
        [ul * v1[:, :64] + uh * v1[:, 64:],
         ul * v2[:, :64] - uh * v2[:, 64:]], axis=1)


def _prep_body(x_ref, rel_ref, loop_ref, win_ref, wout_ref, wloop_ref,
               wrel_ref, f_ref, fd_ref, p_ref,
               xh_ref, bhd_ref, w2in_ref, w2out_ref, w2loop_ref, relout_ref):
    p = p_ref[...]
    xp = jnp.concatenate(
        [x_ref[...], jnp.zeros((NP - N, D), jnp.float32)], axis=0)
    relf = jnp.concatenate(
        [rel_ref[...], loop_ref[...],
         jnp.zeros((RELP - 201, D), jnp.float32)], axis=0)
    xh_ref[...] = _dot(xp, f_ref[...])
    bhd_ref[...] = _dot(relf, fd_ref[...])
    w2in_ref[...] = _dot(p, win_ref[...])
    w2out_ref[...] = _dot(p, wout_ref[...])
    w2loop_ref[...] = _dot(p, wloop_ref[...])
    relout_ref[...] = _dot(relf, wrel_ref[...])[:200]


def _prep(x, rel_embed, loop_rel, w_in, w_out, w_loop, w_rel,
          fmat, fdmat, pmat):
    return pl.pallas_call(
        _prep_body,
        out_shape=[
            jax.ShapeDtypeStruct((NP, REC), jnp.float32),
            jax.ShapeDtypeStruct((RELP, VREC), jnp.float32),
            jax.ShapeDtypeStruct((REC, D), jnp.float32),
            jax.ShapeDtypeStruct((REC, D), jnp.float32),
            jax.ShapeDtypeStruct((REC, D), jnp.float32),
            jax.ShapeDtypeStruct((200, D), jnp.float32),
        ],
    )(x, rel_embed, loop_rel, w_in, w_out, w_loop, w_rel, fmat, fdmat, pmat)


def _packidx_body(ei_ref, et_ref, rows2_ref, idx3_ref):
    rows = ei_ref[0]
    cols = ei_ref[1]
    ts = et_ref[...]
    padi = jnp.full((NEP - NE,), NP - 1, jnp.int32)
    padt = jnp.zeros((NEP - NE,), jnp.int32)
    r_in = jnp.concatenate([rows[:NE], padi])
    r_out = jnp.concatenate([rows[NE:], padi])
    c_in = jnp.concatenate([cols[:NE], padi])
    c_out = jnp.concatenate([cols[NE:] + NP, padi + NP])
    t_in = jnp.concatenate([ts[:NE], padt])
    t_out = jnp.concatenate([ts[NE:], padt])
    rows2_ref[0, :] = r_in
    rows2_ref[1, :] = r_out
    idx3_ref[0, 0, :] = r_in
    idx3_ref[0, 1, :] = c_in
    idx3_ref[0, 2, :] = t_in
    idx3_ref[1, 0, :] = r_out
    idx3_ref[1, 1, :] = c_out
    idx3_ref[1, 2, :] = t_out


def _packidx(edge_index, edge_type):
    return pl.pallas_call(
        _packidx_body,
        out_shape=[
            jax.ShapeDtypeStruct((NC, NEP), jnp.int32),
            jax.ShapeDtypeStruct((NC, 3, NEP), jnp.int32),
        ],
    )(edge_index, edge_type)


def _deg_body(rows_ref, out_ref, ones_v, rbuf, deg_sp):
    c = lax.axis_index("c")
    s = lax.axis_index("s")
    one16 = jnp.where(lax.iota(jnp.int32, 16) == 0,
                      jnp.full((16,), 1.0, jnp.float32),
                      jnp.zeros((16,), jnp.float32))
    zero16 = jnp.zeros((16,), jnp.float32)
    for j in range(BB):
        for k in range(REC // 16):
            ones_v[j, pl.ds(16 * k, 16)] = zero16
    for q in range(RPT // BB):
        pltpu.sync_copy(ones_v, deg_sp.at[pl.ds(s * RPT + q * BB, BB)])
    for j in range(BB):
        ones_v[j, pl.ds(0, 16)] = one16
    plsc.subcore_barrier()

    def blk(b, carry):
        off = s * EPT + b * BB
        pltpu.sync_copy(rows_ref.at[c, pl.ds(off, BB)], rbuf)
        pltpu.sync_copy(ones_v, deg_sp.at[rbuf], add=True)
        return carry

    lax.fori_loop(0, NBLK, blk, 0)
    plsc.subcore_barrier()
    pltpu.sync_copy(deg_sp.at[pl.ds(s * RPT, RPT)],
                    out_ref.at[c, pl.ds(s * RPT, RPT)])


def _degrees(rows2):
    mesh = plsc.VectorSubcoreMesh(core_axis_name="c", subcore_axis_name="s")
    return pl.kernel(
        _deg_body,
        out_type=jax.ShapeDtypeStruct((NC, NP, REC), jnp.float32),
        mesh=mesh,
        scratch_types=[
            pltpu.VMEM((BB, REC), jnp.float32),
            pltpu.VMEM((BB,), jnp.int32),
            pltpu.VMEM_SHARED((NP, REC), jnp.float32),
        ],
    )(rows2)


def _scale_body(xh_ref, degp_ref, y_ref):
    xh = xh_ref[...]
    for c in range(NC):
        deg = degp_ref[c][:, 0:1]
        dinv = jnp.where(deg > 0, lax.rsqrt(deg), 0.0)
        y_ref[pl.ds(c * NP, NP), :] = xh * dinv


def _scale(xh, degp):
    return pl.pallas_call(
        _scale_body,
        out_shape=jax.ShapeDtypeStruct((NC * NP, REC), jnp.float32),
    )(xh, degp)


def _agg_body(y_ref, idx_ref, bhd_ref, out_ref,
              ybuf, bbuf, rbuf, cibuf, tbuf, a_sp):
    c = lax.axis_index("c")
    s = lax.axis_index("s")
    zero16 = jnp.zeros((16,), jnp.float32)
    for j in range(BB):
        for k in range(REC // 16):
            ybuf[j, pl.ds(16 * k, 16)] = zero16
    for q in range(RPT // BB):
        pltpu.sync_copy(ybuf, a_sp.at[pl.ds(s * RPT + q * BB, BB)])
    plsc.subcore_barrier()

    def blk(b, carry):
        off = s * EPT + b * BB
        pltpu.sync_copy(idx_ref.at[c, 0, pl.ds(off, BB)], rbuf)
        pltpu.sync_copy(idx_ref.at[c, 1, pl.ds(off, BB)], cibuf)
        pltpu.sync_copy(idx_ref.at[c, 2, pl.ds(off, BB)], tbuf)
        pltpu.sync_copy(y_ref.at[cibuf], ybuf)
        pltpu.sync_copy(bhd_ref.at[tbuf], bbuf)
        for j in range(BB):
            for k in range(4):
                ul = ybuf[j, pl.ds(16 * k, 16)]
                uh = ybuf[j, pl.ds(64 + 16 * k, 16)]
                v1l = bbuf[j, pl.ds(16 * k, 16)]
                v1h = bbuf[j, pl.ds(64 + 16 * k, 16)]
                v2l = bbuf[j, pl.ds(128 + 16 * k, 16)]
                v2h = bbuf[j, pl.ds(192 + 16 * k, 16)]
                ybuf[j, pl.ds(16 * k, 16)] = ul * v1l + uh * v1h
                ybuf[j, pl.ds(64 + 16 * k, 16)] = ul * v2l - uh * v2h
        pltpu.sync_copy(ybuf, a_sp.at[rbuf], add=True)
        return carry

    lax.fori_loop(0, NBLK, blk, 0)
    plsc.subcore_barrier()
    pltpu.sync_copy(a_sp.at[pl.ds(s * RPT, RPT)],
                    out_ref.at[c, pl.ds(s * RPT, RPT)])


def _aggregate(yflat, idx3, bhd):
    mesh = plsc.VectorSubcoreMesh(core_axis_name="c", subcore_axis_name="s")
    return pl.kernel(
        _agg_body,
        out_type=jax.ShapeDtypeStruct((NC, NP, REC), jnp.float32),
        mesh=mesh,
        scratch_types=[
            pltpu.VMEM((BB, REC), jnp.float32),
            pltpu.VMEM((BB, VREC), jnp.float32),
            pltpu.VMEM((BB,), jnp.int32),
            pltpu.VMEM((BB,), jnp.int32),
            pltpu.VMEM((BB,), jnp.int32),
            pltpu.VMEM_SHARED((NP, REC), jnp.float32),
        ],
    )(yflat, idx3, bhd)


GB = 8
NBR = NP // GB


def _combine_body(ain_ref, aout_ref, xh_ref, bhd_ref, w2in_ref, w2out_ref,
                  w2loop_ref, degin_ref, degout_ref, bias_ref,
                  msg_ref, stats_ref):
    i = pl.program_id(0)

    def dinv(deg_ref):
        deg = deg_ref[0][:, 0:1]
        return jnp.where(deg > 0, lax.rsqrt(deg), 0.0)

    m_in = _dot(ain_ref[0], w2in_ref[...]) * dinv(degin_ref)
    m_out = _dot(aout_ref[0], w2out_ref[...]) * dinv(degout_ref)

    lrec = bhd_ref[200:201, :]
    sl = _cplx_dense(xh_ref[...], lrec[:, :REC], lrec[:, REC:])
    m_loop = _dot(sl, w2loop_ref[...])

    msg = (m_in + m_out + m_loop) * (1.0 / 3.0) + bias_ref[...][None, :]
    msg_ref[...] = msg

    rowid = i * NBR + lax.broadcasted_iota(jnp.int32, (NBR, D), 0)
    msgv = jnp.where(rowid < N, msg, 0.0)
    s1 = jnp.sum(msgv, axis=0, keepdims=True)
    s2 = jnp.sum(msgv * msgv, axis=0, keepdims=True)
    stats_ref[...] = jnp.concatenate(
        [s1, s2, jnp.zeros((6, D), jnp.float32)], axis=0)[None]


def _combine(a2, xh, bhd, w2in, w2out, w2loop, degp, bias):
    full = lambda *dims: pl.BlockSpec(dims, lambda i: (0,) * len(dims))
    return pl.pallas_call(
        _combine_body,
        grid=(GB,),
        in_specs=[
            pl.BlockSpec((1, NBR, REC), lambda i: (0, i, 0)),
            pl.BlockSpec((1, NBR, REC), lambda i: (1, i, 0)),
            pl.BlockSpec((NBR, REC), lambda i: (i, 0)),
            full(RELP, VREC),
            full(REC, D),
            full(REC, D),
            full(REC, D),
            pl.BlockSpec((1, NBR, REC), lambda i: (0, i, 0)),
            pl.BlockSpec((1, NBR, REC), lambda i: (1, i, 0)),
            full(D),
        ],
        out_specs=[
            pl.BlockSpec((NBR, D), lambda i: (i, 0)),
            pl.BlockSpec((1, 8, D), lambda i: (i, 0, 0)),
        ],
        out_shape=[
            jax.ShapeDtypeStruct((NP, D), jnp.float32),
            jax.ShapeDtypeStruct((GB, 8, D), jnp.float32),
        ],
    )(a2, a2, xh, bhd, w2in, w2out, w2loop, degp, degp, bias)


def _bn_body(msg_ref, stats_ref, gamma_ref, beta_ref, out_ref):
    mean = jnp.sum(stats_ref[:, 0, :], axis=0, keepdims=True) * (1.0 / N)
    ex2 = jnp.sum(stats_ref[:, 1, :], axis=0, keepdims=True) * (1.0 / N)
    var = ex2 - mean * mean
    inv = lax.rsqrt(var + 1e-5)
    out_ref[...] = ((msg_ref[...] - mean) * inv * gamma_ref[...][None, :]
                    + beta_ref[...][None, :])


GB2 = 10
NBR2 = N // GB2


def _bn(msg, stats, gamma, beta):
    return pl.pallas_call(
        _bn_body,
        grid=(GB2,),
        in_specs=[
            pl.BlockSpec((NBR2, D), lambda i: (i, 0)),
            pl.BlockSpec((GB, 8, D), lambda i: (0, 0, 0)),
            pl.BlockSpec((D,), lambda i: (0,)),
            pl.BlockSpec((D,), lambda i: (0,)),
        ],
        out_specs=pl.BlockSpec((NBR2, D), lambda i: (i, 0)),
        out_shape=jax.ShapeDtypeStruct((N, D), jnp.float32),
    )(msg, stats, gamma, beta)


def kernel(x, edge_index, edge_type, rel_embed, w_loop, w_in, w_out, w_rel,
           loop_rel, bias, bn_gamma, bn_beta):
    fmat = jnp.asarray(_F2)
    fdmat = jnp.asarray(_FD)
    pmat = jnp.asarray(_P2)

    rows2, idx3 = _packidx(edge_index, edge_type)
    xh, bhd, w2in, w2out, w2loop, relout = _prep(
        x, rel_embed, loop_rel, w_in, w_out, w_loop, w_rel,
        fmat, fdmat, pmat)
    degp = _degrees(rows2)
    yflat = _scale(xh, degp)
    a2 = _aggregate(yflat, idx3, bhd)
    msg, stats = _combine(a2, xh, bhd, w2in, w2out, w2loop, degp, bias)
    out = _bn(msg, stats, bn_gamma, bn_beta)
    return out, relout


# speedup vs baseline: 10.3301x; 2.5952x over previous
import numpy as np
import jax
import jax.numpy as jnp
from jax import lax
from jax.experimental import pallas as pl
from jax.experimental.pallas import tpu as pltpu
from jax.experimental.pallas import tpu_sc as plsc

N = 10000
D = 128
M = 65
REC = 128
VREC = 256
NP = 10240
RELP = 208
NE = 160000
NEP = 163840
NC = 2
NS = 16
EPT = NEP // NS
BB = 64
NBLK = EPT // BB
BA = 128
NBLKA = EPT // BA
RPT = NP // NS

_ang = 2.0 * np.pi * np.outer(np.arange(D), np.arange(M)) / D
_cos = np.cos(_ang)
_sin = np.sin(_ang)
_wj = np.full((M,), 2.0)
_wj[0] = 1.0
_wj[-1] = 1.0

_F2 = np.zeros((D, REC), np.float32)
_F2[:, :64] = _cos[:, :64]
_F2[:, 64] = _cos[:, 64]
_F2[:, 65:] = -_sin[:, 1:64]
_FB = np.zeros((D, REC), np.float32)
_FB[:, :64] = -_sin[:, :64]
_FB[:, 64] = _cos[:, 64]
_FB[:, 65:] = _cos[:, 1:64]
_FD = np.concatenate([_F2, _FB], axis=1)

_Pr = (_wj[:, None] * _cos.T) / D
_Pi = -(_wj[:, None] * _sin.T) / D
_P2 = np.zeros((REC, D), np.float32)
_P2[0] = _Pr[0]
_P2[1:64] = _Pr[1:64]
_P2[64] = _Pr[0] - _Pr[64]
_P2[65:] = _Pi[1:64]

_HI = jax.lax.Precision.HIGHEST


def _dot(a, b):
    return jnp.dot(a, b, preferred_element_type=jnp.float32, precision=_HI)


def _cplx_dense(u, v1, v2):
    ul, uh = u[:, :64], u[:, 64:]
    return jnp.concatenate(
        [ul * v1[:, :64] + uh * v1[:, 64:],
         ul * v2[:, :64] - uh * v2[:, 64:]], axis=1)


def _prep_body(x_ref, rel_ref, loop_ref, win_ref, wout_ref, wloop_ref,
               wrel_ref, f_ref, fd_ref, p_ref,
               xh_ref, bh1_ref, ldual_ref, w2in_ref, w2out_ref, w2loop_ref,
               relout_ref):
    p = p_ref[...]
    xp = jnp.concatenate(
        [x_ref[...], jnp.zeros((NP - N, D), jnp.float32)], axis=0)
    relf = jnp.concatenate(
        [rel_ref[...], loop_ref[...],
         jnp.zeros((RELP - 201, D), jnp.float32)], axis=0)
    xh_ref[...] = _dot(xp, f_ref[...])
    bh1_ref[...] = _dot(relf, f_ref[...])
    ldual_ref[...] = _dot(relf[200:208], fd_ref[...])
    w2in_ref[...] = _dot(p, win_ref[...])
    w2out_ref[...] = _dot(p, wout_ref[...])
    w2loop_ref[...] = _dot(p, wloop_ref[...])
    relout_ref[...] = _dot(relf, wrel_ref[...])[:200]


def _prep(x, rel_embed, loop_rel, w_in, w_out, w_loop, w_rel,
          fmat, fdmat, pmat):
    return pl.pallas_call(
        _prep_body,
        out_shape=[
            jax.ShapeDtypeStruct((NP, REC), jnp.float32),
            jax.ShapeDtypeStruct((RELP, REC), jnp.float32),
            jax.ShapeDtypeStruct((8, VREC), jnp.float32),
            jax.ShapeDtypeStruct((REC, D), jnp.float32),
            jax.ShapeDtypeStruct((REC, D), jnp.float32),
            jax.ShapeDtypeStruct((REC, D), jnp.float32),
            jax.ShapeDtypeStruct((200, D), jnp.float32),
        ],
    )(x, rel_embed, loop_rel, w_in, w_out, w_loop, w_rel, fmat, fdmat, pmat)


def _packidx_body(ei_ref, et_ref, rows2_ref, idx3_ref):
    rows = ei_ref[0]
    cols = ei_ref[1]
    ts = et_ref[...]
    padi = jnp.full((NEP - NE,), NP - 1, jnp.int32)
    padt = jnp.zeros((NEP - NE,), jnp.int32)
    r_in = jnp.concatenate([rows[:NE], padi])
    r_out = jnp.concatenate([rows[NE:], padi])
    c_in = jnp.concatenate([cols[:NE], padi])
    c_out = jnp.concatenate([cols[NE:] + NP, padi + NP])
    t_in = jnp.concatenate([ts[:NE], padt])
    t_out = jnp.concatenate([ts[NE:], padt])
    rows2_ref[0, :] = r_in
    rows2_ref[1, :] = r_out
    idx3_ref[0, 0, :] = r_in
    idx3_ref[0, 1, :] = c_in
    idx3_ref[0, 2, :] = t_in
    idx3_ref[1, 0, :] = r_out
    idx3_ref[1, 1, :] = c_out
    idx3_ref[1, 2, :] = t_out


def _packidx(edge_index, edge_type):
    return pl.pallas_call(
        _packidx_body,
        out_shape=[
            jax.ShapeDtypeStruct((NC, NEP), jnp.int32),
            jax.ShapeDtypeStruct((NC, 3, NEP), jnp.int32),
        ],
    )(edge_index, edge_type)


def _deg_body(rows_ref, out_ref, ones_v, rbuf, deg_sp):
    c = lax.axis_index("c")
    s = lax.axis_index("s")
    one16 = jnp.where(lax.iota(jnp.int32, 16) == 0,
                      jnp.full((16,), 1.0, jnp.float32),
                      jnp.zeros((16,), jnp.float32))
    zero16 = jnp.zeros((16,), jnp.float32)
    for j in range(BB):
        for k in range(REC // 16):
            ones_v[j, pl.ds(16 * k, 16)] = zero16
    for q in range(RPT // BB):
        pltpu.sync_copy(ones_v, deg_sp.at[pl.ds(s * RPT + q * BB, BB)])
    for j in range(BB):
        ones_v[j, pl.ds(0, 16)] = one16
    plsc.subcore_barrier()

    def blk(b, carry):
        off = s * EPT + b * BB
        pltpu.sync_copy(rows_ref.at[c, pl.ds(off, BB)], rbuf)
        pltpu.sync_copy(ones_v, deg_sp.at[rbuf], add=True)
        return carry

    lax.fori_loop(0, NBLK, blk, 0)
    plsc.subcore_barrier()
    pltpu.sync_copy(deg_sp.at[pl.ds(s * RPT, RPT)],
                    out_ref.at[c, pl.ds(s * RPT, RPT)])


def _degrees(rows2):
    mesh = plsc.VectorSubcoreMesh(core_axis_name="c", subcore_axis_name="s")
    return pl.kernel(
        _deg_body,
        out_type=jax.ShapeDtypeStruct((NC, NP, REC), jnp.float32),
        mesh=mesh,
        scratch_types=[
            pltpu.VMEM((BB, REC), jnp.float32),
            pltpu.VMEM((BB,), jnp.int32),
            pltpu.VMEM_SHARED((NP, REC), jnp.float32),
        ],
    )(rows2)


def _scale_body(xh_ref, degp_ref, y_ref):
    xh = xh_ref[...]
    for c in range(NC):
        deg = degp_ref[c][:, 0:1]
        dinv = jnp.where(deg > 0, lax.rsqrt(deg), 0.0)
        y_ref[pl.ds(c * NP, NP), :] = xh * dinv


def _scale(xh, degp):
    return pl.pallas_call(
        _scale_body,
        out_shape=jax.ShapeDtypeStruct((NC * NP, REC), jnp.float32),
    )(xh, degp)


def _agg_body(y_ref, idx_ref, bh1_ref, out_ref,
              ybuf, bbuf, ibuf, a_sp, bh_sp):
    c = lax.axis_index("c")
    s = lax.axis_index("s")
    zero16 = jnp.zeros((16,), jnp.float32)
    lane0 = lax.iota(jnp.int32, 16) == 0

    def zrow(j, carry):
        for k in range(REC // 16):
            ybuf[j, pl.ds(16 * k, 16)] = zero16
        return carry

    lax.fori_loop(0, BA, zrow, 0)
    for q in range(RPT // BA):
        pltpu.sync_copy(ybuf, a_sp.at[pl.ds(s * RPT + q * BA, BA)])

    @pl.when(s == 0)
    def _():
        pltpu.sync_copy(bh1_ref, bh_sp)

    plsc.subcore_barrier()

    def edge(j, carry):
        for k in range(4):
            ul = ybuf[j, pl.ds(16 * k, 16)]
            uh = ybuf[j, pl.ds(64 + 16 * k, 16)]
            vl = bbuf[j, pl.ds(16 * k, 16)]
            vh = bbuf[j, pl.ds(64 + 16 * k, 16)]
            if k == 0:
                v2l = jnp.where(lane0, 0.0, vh)
                v2h = jnp.where(lane0, vh, vl)
            else:
                v2l, v2h = vh, vl
            ybuf[j, pl.ds(16 * k, 16)] = ul * vl + uh * vh
            ybuf[j, pl.ds(64 + 16 * k, 16)] = ul * v2l - uh * v2h
        return carry

    def blk(b, carry):
        off = s * EPT + b * BA
        pltpu.sync_copy(idx_ref.at[c, :, pl.ds(off, BA)], ibuf)
        pltpu.sync_copy(y_ref.at[ibuf.at[1]], ybuf)
        pltpu.sync_copy(bh_sp.at[ibuf.at[2]], bbuf)
        lax.fori_loop(0, BA, edge, 0)
        pltpu.sync_copy(ybuf, a_sp.at[ibuf.at[0]], add=True)
        return carry

    lax.fori_loop(0, NBLKA, blk, 0)
    plsc.subcore_barrier()
    pltpu.sync_copy(a_sp.at[pl.ds(s * RPT, RPT)],
                    out_ref.at[c, pl.ds(s * RPT, RPT)])


def _aggregate(yflat, idx3, bh1):
    mesh = plsc.VectorSubcoreMesh(core_axis_name="c", subcore_axis_name="s")
    return pl.kernel(
        _agg_body,
        out_type=jax.ShapeDtypeStruct((NC, NP, REC), jnp.float32),
        mesh=mesh,
        scratch_types=[
            pltpu.VMEM((BA, REC), jnp.float32),
            pltpu.VMEM((BA, REC), jnp.float32),
            pltpu.VMEM((3, BA), jnp.int32),
            pltpu.VMEM_SHARED((NP, REC), jnp.float32),
            pltpu.VMEM_SHARED((RELP, REC), jnp.float32),
        ],
    )(yflat, idx3, bh1)


GB = 8
NBR = NP // GB


def _combine_body(ain_ref, aout_ref, xh_ref, ldual_ref, w2in_ref, w2out_ref,
                  w2loop_ref, degin_ref, degout_ref, bias_ref,
                  msg_ref, stats_ref):
    i = pl.program_id(0)

    def dinv(deg_ref):
        deg = deg_ref[0][:, 0:1]
        return jnp.where(deg > 0, lax.rsqrt(deg), 0.0)

    m_in = _dot(ain_ref[0], w2in_ref[...]) * dinv(degin_ref)
    m_out = _dot(aout_ref[0], w2out_ref[...]) * dinv(degout_ref)

    lrec = ldual_ref[0:1, :]
    sl = _cplx_dense(xh_ref[...], lrec[:, :REC], lrec[:, REC:])
    m_loop = _dot(sl, w2loop_ref[...])

    msg = (m_in + m_out + m_loop) * (1.0 / 3.0) + bias_ref[...][None, :]
    msg_ref[...] = msg

    rowid = i * NBR + lax.broadcasted_iota(jnp.int32, (NBR, D), 0)
    msgv = jnp.where(rowid < N, msg, 0.0)
    s1 = jnp.sum(msgv, axis=0, keepdims=True)
    s2 = jnp.sum(msgv * msgv, axis=0, keepdims=True)
    stats_ref[...] = jnp.concatenate(
        [s1, s2, jnp.zeros((6, D), jnp.float32)], axis=0)[None]


def _combine(a2, xh, ldual, w2in, w2out, w2loop, degp, bias):
    full = lambda *dims: pl.BlockSpec(dims, lambda i: (0,) * len(dims))
    return pl.pallas_call(
        _combine_body,
        grid=(GB,),
        in_specs=[
            pl.BlockSpec((1, NBR, REC), lambda i: (0, i, 0)),
            pl.BlockSpec((1, NBR, REC), lambda i: (1, i, 0)),
            pl.BlockSpec((NBR, REC), lambda i: (i, 0)),
            full(8, VREC),
            full(REC, D),
            full(REC, D),
            full(REC, D),
            pl.BlockSpec((1, NBR, REC), lambda i: (0, i, 0)),
            pl.BlockSpec((1, NBR, REC), lambda i: (1, i, 0)),
            full(D),
        ],
        out_specs=[
            pl.BlockSpec((NBR, D), lambda i: (i, 0)),
            pl.BlockSpec((1, 8, D), lambda i: (i, 0, 0)),
        ],
        out_shape=[
            jax.ShapeDtypeStruct((NP, D), jnp.float32),
            jax.ShapeDtypeStruct((GB, 8, D), jnp.float32),
        ],
    )(a2, a2, xh, ldual, w2in, w2out, w2loop, degp, degp, bias)


def _bn_body(msg_ref, stats_ref, gamma_ref, beta_ref, out_ref):
    mean = jnp.sum(stats_ref[:, 0, :], axis=0, keepdims=True) * (1.0 / N)
    ex2 = jnp.sum(stats_ref[:, 1, :], axis=0, keepdims=True) * (1.0 / N)
    var = ex2 - mean * mean
    inv = lax.rsqrt(var + 1e-5)
    out_ref[...] = ((msg_ref[...] - mean) * inv * gamma_ref[...][None, :]
                    + beta_ref[...][None, :])


GB2 = 10
NBR2 = N // GB2


def _bn(msg, stats, gamma, beta):
    return pl.pallas_call(
        _bn_body,
        grid=(GB2,),
        in_specs=[
            pl.BlockSpec((NBR2, D), lambda i: (i, 0)),
            pl.BlockSpec((GB, 8, D), lambda i: (0, 0, 0)),
            pl.BlockSpec((D,), lambda i: (0,)),
            pl.BlockSpec((D,), lambda i: (0,)),
        ],
        out_specs=pl.BlockSpec((NBR2, D), lambda i: (i, 0)),
        out_shape=jax.ShapeDtypeStruct((N, D), jnp.float32),
    )(msg, stats, gamma, beta)


def kernel(x, edge_index, edge_type, rel_embed, w_loop, w_in, w_out, w_rel,
           loop_rel, bias, bn_gamma, bn_beta):
    fmat = jnp.asarray(_F2)
    fdmat = jnp.asarray(_FD)
    pmat = jnp.asarray(_P2)

    rows2, idx3 = _packidx(edge_index, edge_type)
    xh, bh1, ldual, w2in, w2out, w2loop, relout = _prep(
        x, rel_embed, loop_rel, w_in, w_out, w_loop, w_rel,
        fmat, fdmat, pmat)
    degp = _degrees(rows2)
    yflat = _scale(xh, degp)
    a2 = _aggregate(yflat, idx3, bh1)
    msg, stats = _combine(a2, xh, ldual, w2in, w2out, w2loop, degp, bias)
    out = _bn(msg, stats, bn_gamma, bn_beta)
    return out, relout


# speedup vs baseline: 12.8228x; 1.2413x over previous
import numpy as np
import jax
import jax.numpy as jnp
from jax import lax
from jax.experimental import pallas as pl
from jax.experimental.pallas import tpu as pltpu
from jax.experimental.pallas import tpu_sc as plsc

N = 10000
D = 128
M = 65
REC = 128
VREC = 256
NP = 10240
RELP = 208
NE = 160000
NEP = 163840
NC = 2
NS = 16
EPT = NEP // NS
BB = 64
NBLK = EPT // BB
BA = 128
NBLKA = EPT // BA
BH = 64
RPT = NP // NS

_ang = 2.0 * np.pi * np.outer(np.arange(D), np.arange(M)) / D
_cos = np.cos(_ang)
_sin = np.sin(_ang)
_wj = np.full((M,), 2.0)
_wj[0] = 1.0
_wj[-1] = 1.0

_F2 = np.zeros((D, REC), np.float32)
_F2[:, :64] = _cos[:, :64]
_F2[:, 64] = _cos[:, 64]
_F2[:, 65:] = -_sin[:, 1:64]
_FB = np.zeros((D, REC), np.float32)
_FB[:, :64] = -_sin[:, :64]
_FB[:, 64] = _cos[:, 64]
_FB[:, 65:] = _cos[:, 1:64]
_FD = np.concatenate([_F2, _FB], axis=1)

_Pr = (_wj[:, None] * _cos.T) / D
_Pi = -(_wj[:, None] * _sin.T) / D
_P2 = np.zeros((REC, D), np.float32)
_P2[0] = _Pr[0]
_P2[1:64] = _Pr[1:64]
_P2[64] = _Pr[0] - _Pr[64]
_P2[65:] = _Pi[1:64]

_HI = jax.lax.Precision.HIGHEST


def _dot(a, b):
    return jnp.dot(a, b, preferred_element_type=jnp.float32, precision=_HI)


def _cplx_dense(u, v1, v2):
    ul, uh = u[:, :64], u[:, 64:]
    return jnp.concatenate(
        [ul * v1[:, :64] + uh * v1[:, 64:],
         ul * v2[:, :64] - uh * v2[:, 64:]], axis=1)


def _prep_body(x_ref, rel_ref, loop_ref, win_ref, wout_ref, wloop_ref,
               wrel_ref, f_ref, fd_ref, p_ref,
               xh_ref, bh1_ref, ldual_ref, w2in_ref, w2out_ref, w2loop_ref,
               relout_ref):
    p = p_ref[...]
    xp = jnp.concatenate(
        [x_ref[...], jnp.zeros((NP - N, D), jnp.float32)], axis=0)
    relf = jnp.concatenate(
        [rel_ref[...], loop_ref[...],
         jnp.zeros((RELP - 201, D), jnp.float32)], axis=0)
    xh_ref[...] = _dot(xp, f_ref[...])
    bh1_ref[...] = _dot(relf, f_ref[...])
    ldual_ref[...] = _dot(relf[200:208], fd_ref[...])
    w2in_ref[...] = _dot(p, win_ref[...])
    w2out_ref[...] = _dot(p, wout_ref[...])
    w2loop_ref[...] = _dot(p, wloop_ref[...])
    relout_ref[...] = _dot(relf, wrel_ref[...])[:200]


def _prep(x, rel_embed, loop_rel, w_in, w_out, w_loop, w_rel,
          fmat, fdmat, pmat):
    return pl.pallas_call(
        _prep_body,
        out_shape=[
            jax.ShapeDtypeStruct((NP, REC), jnp.float32),
            jax.ShapeDtypeStruct((RELP, REC), jnp.float32),
            jax.ShapeDtypeStruct((8, VREC), jnp.float32),
            jax.ShapeDtypeStruct((REC, D), jnp.float32),
            jax.ShapeDtypeStruct((REC, D), jnp.float32),
            jax.ShapeDtypeStruct((REC, D), jnp.float32),
            jax.ShapeDtypeStruct((200, D), jnp.float32),
        ],
    )(x, rel_embed, loop_rel, w_in, w_out, w_loop, w_rel, fmat, fdmat, pmat)


def _packidx_body(ei_ref, et_ref, rows2_ref, idx3_ref):
    rows = ei_ref[0]
    cols = ei_ref[1]
    ts = et_ref[...]
    padi = jnp.full((NEP - NE,), NP - 1, jnp.int32)
    padt = jnp.zeros((NEP - NE,), jnp.int32)
    r_in = jnp.concatenate([rows[:NE], padi])
    r_out = jnp.concatenate([rows[NE:], padi])
    c_in = jnp.concatenate([cols[:NE], padi])
    c_out = jnp.concatenate([cols[NE:] + NP, padi + NP])
    t_in = jnp.concatenate([ts[:NE], padt])
    t_out = jnp.concatenate([ts[NE:], padt])
    rows2_ref[0, :] = r_in
    rows2_ref[1, :] = r_out
    idx3_ref[0, 0, :] = r_in
    idx3_ref[0, 1, :] = c_in
    idx3_ref[0, 2, :] = t_in
    idx3_ref[1, 0, :] = r_out
    idx3_ref[1, 1, :] = c_out
    idx3_ref[1, 2, :] = t_out


def _packidx(edge_index, edge_type):
    return pl.pallas_call(
        _packidx_body,
        out_shape=[
            jax.ShapeDtypeStruct((NC, NEP), jnp.int32),
            jax.ShapeDtypeStruct((NC, 3, NEP), jnp.int32),
        ],
    )(edge_index, edge_type)


def _deg_body(rows_ref, out_ref, ones_v, rbuf, deg_sp):
    c = lax.axis_index("c")
    s = lax.axis_index("s")
    one16 = jnp.where(lax.iota(jnp.int32, 16) == 0,
                      jnp.full((16,), 1.0, jnp.float32),
                      jnp.zeros((16,), jnp.float32))
    zero16 = jnp.zeros((16,), jnp.float32)
    for j in range(BB):
        for k in range(REC // 16):
            ones_v[j, pl.ds(16 * k, 16)] = zero16
    for q in range(RPT // BB):
        pltpu.sync_copy(ones_v, deg_sp.at[pl.ds(s * RPT + q * BB, BB)])
    for j in range(BB):
        ones_v[j, pl.ds(0, 16)] = one16
    plsc.subcore_barrier()

    def blk(b, carry):
        off = s * EPT + b * BB
        pltpu.sync_copy(rows_ref.at[c, pl.ds(off, BB)], rbuf)
        pltpu.sync_copy(ones_v, deg_sp.at[rbuf], add=True)
        return carry

    lax.fori_loop(0, NBLK, blk, 0)
    plsc.subcore_barrier()
    pltpu.sync_copy(deg_sp.at[pl.ds(s * RPT, RPT)],
                    out_ref.at[c, pl.ds(s * RPT, RPT)])


def _degrees(rows2):
    mesh = plsc.VectorSubcoreMesh(core_axis_name="c", subcore_axis_name="s")
    return pl.kernel(
        _deg_body,
        out_type=jax.ShapeDtypeStruct((NC, NP, REC), jnp.float32),
        mesh=mesh,
        scratch_types=[
            pltpu.VMEM((BB, REC), jnp.float32),
            pltpu.VMEM((BB,), jnp.int32),
            pltpu.VMEM_SHARED((NP, REC), jnp.float32),
        ],
    )(rows2)


def _scale_body(xh_ref, degp_ref, y_ref):
    xh = xh_ref[...]
    for c in range(NC):
        deg = degp_ref[c][:, 0:1]
        dinv = jnp.where(deg > 0, lax.rsqrt(deg), 0.0)
        y_ref[pl.ds(c * NP, NP), :] = xh * dinv


def _scale(xh, degp):
    return pl.pallas_call(
        _scale_body,
        out_shape=jax.ShapeDtypeStruct((NC * NP, REC), jnp.float32),
    )(xh, degp)


def _agg_body(y_ref, idx_ref, bh1_ref, out_ref,
              ybuf0, ybuf1, bbuf, ibuf0, ibuf1, sg0, sg1, a_sp, bh_sp):
    c = lax.axis_index("c")
    s = lax.axis_index("s")
    zero16 = jnp.zeros((16,), jnp.float32)
    lane0 = lax.iota(jnp.int32, 16) == 0

    def zrow(j, carry):
        for k in range(REC // 16):
            ybuf0[j, pl.ds(16 * k, 16)] = zero16
        return carry

    lax.fori_loop(0, BA, zrow, 0)
    for q in range(RPT // BA):
        pltpu.sync_copy(ybuf0, a_sp.at[pl.ds(s * RPT + q * BA, BA)])

    @pl.when(s == 0)
    def _():
        pltpu.sync_copy(bh1_ref, bh_sp)

    plsc.subcore_barrier()

    def make_edge(ybuf, off):
        def edge(j, carry):
            for k in range(4):
                ul = ybuf[off + j, pl.ds(16 * k, 16)]
                uh = ybuf[off + j, pl.ds(64 + 16 * k, 16)]
                vl = bbuf[j, pl.ds(16 * k, 16)]
                vh = bbuf[j, pl.ds(64 + 16 * k, 16)]
                if k == 0:
                    v2l = jnp.where(lane0, 0.0, vh)
                    v2h = jnp.where(lane0, vh, vl)
                else:
                    v2l, v2h = vh, vl
                ybuf[off + j, pl.ds(16 * k, 16)] = ul * vl + uh * vh
                ybuf[off + j, pl.ds(64 + 16 * k, 16)] = ul * v2l - uh * v2h
            return carry
        return edge

    base = s * EPT

    def phase(b, ybuf, ibuf, sem, ibuf_n, ybuf_n, sem_n):
        @pl.when(b + 1 < NBLKA)
        def _():
            pltpu.sync_copy(idx_ref.at[c, :, pl.ds(base + (b + 1) * BA, BA)],
                            ibuf_n)
            pltpu.async_copy(y_ref.at[ibuf_n.at[1]], ybuf_n, sem_n)

        pltpu.make_async_copy(y_ref.at[ibuf.at[1]], ybuf, sem).wait()
        for h in range(BA // BH):
            pltpu.sync_copy(bh_sp.at[ibuf.at[2, pl.ds(BH * h, BH)]], bbuf)
            lax.fori_loop(0, BH, make_edge(ybuf, BH * h), 0)
        pltpu.sync_copy(ybuf, a_sp.at[ibuf.at[0]], add=True)

    pltpu.sync_copy(idx_ref.at[c, :, pl.ds(base, BA)], ibuf0)
    pltpu.async_copy(y_ref.at[ibuf0.at[1]], ybuf0, sg0)

    def blk2(b2, carry):
        phase(2 * b2, ybuf0, ibuf0, sg0, ibuf1, ybuf1, sg1)
        phase(2 * b2 + 1, ybuf1, ibuf1, sg1, ibuf0, ybuf0, sg0)
        return carry

    lax.fori_loop(0, NBLKA // 2, blk2, 0)
    plsc.subcore_barrier()
    pltpu.sync_copy(a_sp.at[pl.ds(s * RPT, RPT)],
                    out_ref.at[c, pl.ds(s * RPT, RPT)])


def _aggregate(yflat, idx3, bh1):
    mesh = plsc.VectorSubcoreMesh(core_axis_name="c", subcore_axis_name="s")
    return pl.kernel(
        _agg_body,
        out_type=jax.ShapeDtypeStruct((NC, NP, REC), jnp.float32),
        mesh=mesh,
        scratch_types=[
            pltpu.VMEM((BA, REC), jnp.float32),
            pltpu.VMEM((BA, REC), jnp.float32),
            pltpu.VMEM((BH, REC), jnp.float32),
            pltpu.VMEM((3, BA), jnp.int32),
            pltpu.VMEM((3, BA), jnp.int32),
            pltpu.SemaphoreType.DMA,
            pltpu.SemaphoreType.DMA,
            pltpu.VMEM_SHARED((NP, REC), jnp.float32),
            pltpu.VMEM_SHARED((RELP, REC), jnp.float32),
        ],
    )(yflat, idx3, bh1)


GB = 8
NBR = NP // GB


def _combine_body(ain_ref, aout_ref, xh_ref, ldual_ref, w2in_ref, w2out_ref,
                  w2loop_ref, degin_ref, degout_ref, bias_ref,
                  msg_ref, stats_ref):
    i = pl.program_id(0)

    def dinv(deg_ref):
        deg = deg_ref[0][:, 0:1]
        return jnp.where(deg > 0, lax.rsqrt(deg), 0.0)

    m_in = _dot(ain_ref[0], w2in_ref[...]) * dinv(degin_ref)
    m_out = _dot(aout_ref[0], w2out_ref[...]) * dinv(degout_ref)

    lrec = ldual_ref[0:1, :]
    sl = _cplx_dense(xh_ref[...], lrec[:, :REC], lrec[:, REC:])
    m_loop = _dot(sl, w2loop_ref[...])

    msg = (m_in + m_out + m_loop) * (1.0 / 3.0) + bias_ref[...][None, :]
    msg_ref[...] = msg

    rowid = i * NBR + lax.broadcasted_iota(jnp.int32, (NBR, D), 0)
    msgv = jnp.where(rowid < N, msg, 0.0)
    s1 = jnp.sum(msgv, axis=0, keepdims=True)
    s2 = jnp.sum(msgv * msgv, axis=0, keepdims=True)
    stats_ref[...] = jnp.concatenate(
        [s1, s2, jnp.zeros((6, D), jnp.float32)], axis=0)[None]


def _combine(a2, xh, ldual, w2in, w2out, w2loop, degp, bias):
    full = lambda *dims: pl.BlockSpec(dims, lambda i: (0,) * len(dims))
    return pl.pallas_call(
        _combine_body,
        grid=(GB,),
        in_specs=[
            pl.BlockSpec((1, NBR, REC), lambda i: (0, i, 0)),
            pl.BlockSpec((1, NBR, REC), lambda i: (1, i, 0)),
            pl.BlockSpec((NBR, REC), lambda i: (i, 0)),
            full(8, VREC),
            full(REC, D),
            full(REC, D),
            full(REC, D),
            pl.BlockSpec((1, NBR, REC), lambda i: (0, i, 0)),
            pl.BlockSpec((1, NBR, REC), lambda i: (1, i, 0)),
            full(D),
        ],
        out_specs=[
            pl.BlockSpec((NBR, D), lambda i: (i, 0)),
            pl.BlockSpec((1, 8, D), lambda i: (i, 0, 0)),
        ],
        out_shape=[
            jax.ShapeDtypeStruct((NP, D), jnp.float32),
            jax.ShapeDtypeStruct((GB, 8, D), jnp.float32),
        ],
    )(a2, a2, xh, ldual, w2in, w2out, w2loop, degp, degp, bias)


def _bn_body(msg_ref, stats_ref, gamma_ref, beta_ref, out_ref):
    mean = jnp.sum(stats_ref[:, 0, :], axis=0, keepdims=True) * (1.0 / N)
    ex2 = jnp.sum(stats_ref[:, 1, :], axis=0, keepdims=True) * (1.0 / N)
    var = ex2 - mean * mean
    inv = lax.rsqrt(var + 1e-5)
    out_ref[...] = ((msg_ref[...] - mean) * inv * gamma_ref[...][None, :]
                    + beta_ref[...][None, :])


GB2 = 10
NBR2 = N // GB2


def _bn(msg, stats, gamma, beta):
    return pl.pallas_call(
        _bn_body,
        grid=(GB2,),
        in_specs=[
            pl.BlockSpec((NBR2, D), lambda i: (i, 0)),
            pl.BlockSpec((GB, 8, D), lambda i: (0, 0, 0)),
            pl.BlockSpec((D,), lambda i: (0,)),
            pl.BlockSpec((D,), lambda i: (0,)),
        ],
        out_specs=pl.BlockSpec((NBR2, D), lambda i: (i, 0)),
        out_shape=jax.ShapeDtypeStruct((N, D), jnp.float32),
    )(msg, stats, gamma, beta)


def kernel(x, edge_index, edge_type, rel_embed, w_loop, w_in, w_out, w_rel,
           loop_rel, bias, bn_gamma, bn_beta):
    fmat = jnp.asarray(_F2)
    fdmat = jnp.asarray(_FD)
    pmat = jnp.asarray(_P2)

    rows2, idx3 = _packidx(edge_index, edge_type)
    xh, bh1, ldual, w2in, w2out, w2loop, relout = _prep(
        x, rel_embed, loop_rel, w_in, w_out, w_loop, w_rel,
        fmat, fdmat, pmat)
    degp = _degrees(rows2)
    yflat = _scale(xh, degp)
    a2 = _aggregate(yflat, idx3, bh1)
    msg, stats = _combine(a2, xh, ldual, w2in, w2out, w2loop, degp, bias)
    out = _bn(msg, stats, bn_gamma, bn_beta)
    return out, relout


# speedup vs baseline: 14.4436x; 1.1264x over previous
import numpy as np
import jax
import jax.numpy as jnp
from jax import lax
from jax.experimental import pallas as pl
from jax.experimental.pallas import tpu as pltpu
from jax.experimental.pallas import tpu_sc as plsc

N = 10000
D = 128
M = 65
REC = 128
VREC = 256
NP = 10240
RELP = 208
NE = 160000
NEP = 163840
NC = 2
NS = 16
EPT = NEP // NS
BB = 128
NBLK = EPT // BB
BA = 128
NBLKA = EPT // BA
BH = 64
RPT = NP // NS

_ang = 2.0 * np.pi * np.outer(np.arange(D), np.arange(M)) / D
_cos = np.cos(_ang)
_sin = np.sin(_ang)
_wj = np.full((M,), 2.0)
_wj[0] = 1.0
_wj[-1] = 1.0

_F2 = np.zeros((D, REC), np.float32)
_F2[:, :64] = _cos[:, :64]
_F2[:, 64] = _cos[:, 64]
_F2[:, 65:] = -_sin[:, 1:64]
_FB = np.zeros((D, REC), np.float32)
_FB[:, :64] = -_sin[:, :64]
_FB[:, 64] = _cos[:, 64]
_FB[:, 65:] = _cos[:, 1:64]
_FD = np.concatenate([_F2, _FB], axis=1)

_Pr = (_wj[:, None] * _cos.T) / D
_Pi = -(_wj[:, None] * _sin.T) / D
_P2 = np.zeros((REC, D), np.float32)
_P2[0] = _Pr[0]
_P2[1:64] = _Pr[1:64]
_P2[64] = _Pr[0] - _Pr[64]
_P2[65:] = _Pi[1:64]

_HI = jax.lax.Precision.HIGHEST


def _dot(a, b):
    return jnp.dot(a, b, preferred_element_type=jnp.float32, precision=_HI)


def _cplx_dense(u, v1, v2):
    ul, uh = u[:, :64], u[:, 64:]
    return jnp.concatenate(
        [ul * v1[:, :64] + uh * v1[:, 64:],
         ul * v2[:, :64] - uh * v2[:, 64:]], axis=1)


def _prep_body(x_ref, rel_ref, loop_ref, win_ref, wout_ref, wloop_ref,
               wrel_ref, f_ref, fd_ref, p_ref,
               xh_ref, bh1_ref, ldual_ref, w2in_ref, w2out_ref, w2loop_ref,
               relout_ref):
    p = p_ref[...]
    xp = jnp.concatenate(
        [x_ref[...], jnp.zeros((NP - N, D), jnp.float32)], axis=0)
    relf = jnp.concatenate(
        [rel_ref[...], loop_ref[...],
         jnp.zeros((RELP - 201, D), jnp.float32)], axis=0)
    xh_ref[...] = _dot(xp, f_ref[...])
    bh1_ref[...] = _dot(relf, f_ref[...])
    ldual_ref[...] = _dot(relf[200:208], fd_ref[...])
    w2in_ref[...] = _dot(p, win_ref[...])
    w2out_ref[...] = _dot(p, wout_ref[...])
    w2loop_ref[...] = _dot(p, wloop_ref[...])
    relout_ref[...] = _dot(relf, wrel_ref[...])[:200]


def _prep(x, rel_embed, loop_rel, w_in, w_out, w_loop, w_rel,
          fmat, fdmat, pmat):
    return pl.pallas_call(
        _prep_body,
        out_shape=[
            jax.ShapeDtypeStruct((NP, REC), jnp.float32),
            jax.ShapeDtypeStruct((RELP, REC), jnp.float32),
            jax.ShapeDtypeStruct((8, VREC), jnp.float32),
            jax.ShapeDtypeStruct((REC, D), jnp.float32),
            jax.ShapeDtypeStruct((REC, D), jnp.float32),
            jax.ShapeDtypeStruct((REC, D), jnp.float32),
            jax.ShapeDtypeStruct((200, D), jnp.float32),
        ],
    )(x, rel_embed, loop_rel, w_in, w_out, w_loop, w_rel, fmat, fdmat, pmat)


def _packidx_body(ei_ref, et_ref, rows2_ref, idx3_ref):
    rows = ei_ref[0]
    cols = ei_ref[1]
    ts = et_ref[...]
    padi = jnp.full((NEP - NE,), NP - 1, jnp.int32)
    padt = jnp.zeros((NEP - NE,), jnp.int32)
    r_in = jnp.concatenate([rows[:NE], padi])
    r_out = jnp.concatenate([rows[NE:], padi])
    c_in = jnp.concatenate([cols[:NE], padi])
    c_out = jnp.concatenate([cols[NE:] + NP, padi + NP])
    t_in = jnp.concatenate([ts[:NE], padt])
    t_out = jnp.concatenate([ts[NE:], padt])
    rows2_ref[0, :] = r_in
    rows2_ref[1, :] = r_out
    idx3_ref[0, 0, :] = r_in
    idx3_ref[0, 1, :] = c_in
    idx3_ref[0, 2, :] = t_in
    idx3_ref[1, 0, :] = r_out
    idx3_ref[1, 1, :] = c_out
    idx3_ref[1, 2, :] = t_out


def _packidx(edge_index, edge_type):
    return pl.pallas_call(
        _packidx_body,
        out_shape=[
            jax.ShapeDtypeStruct((NC, NEP), jnp.int32),
            jax.ShapeDtypeStruct((NC, 3, NEP), jnp.int32),
        ],
    )(edge_index, edge_type)


def _deg_body(rows_ref, out_ref, ones_v, rbuf0, rbuf1, sd0, sd1, deg_sp):
    c = lax.axis_index("c")
    s = lax.axis_index("s")
    one16 = jnp.where(lax.iota(jnp.int32, 16) == 0,
                      jnp.full((16,), 1.0, jnp.float32),
                      jnp.zeros((16,), jnp.float32))
    zero16 = jnp.zeros((16,), jnp.float32)
    for j in range(BB):
        for k in range(REC // 16):
            ones_v[j, pl.ds(16 * k, 16)] = zero16
    for q in range(RPT // BB):
        pltpu.sync_copy(ones_v, deg_sp.at[pl.ds(s * RPT + q * BB, BB)])
    for j in range(BB):
        ones_v[j, pl.ds(0, 16)] = one16
    plsc.subcore_barrier()

    def phase(b, rbuf, sem):
        @pl.when(b >= 2)
        def _():
            pltpu.make_async_copy(ones_v, deg_sp.at[rbuf], sem).wait()

        pltpu.sync_copy(rows_ref.at[c, pl.ds(s * EPT + b * BB, BB)], rbuf)
        pltpu.async_copy(ones_v, deg_sp.at[rbuf], sem, add=True)

    def blk2(b2, carry):
        phase(2 * b2, rbuf0, sd0)
        phase(2 * b2 + 1, rbuf1, sd1)
        return carry

    lax.fori_loop(0, NBLK // 2, blk2, 0)
    pltpu.make_async_copy(ones_v, deg_sp.at[rbuf0], sd0).wait()
    pltpu.make_async_copy(ones_v, deg_sp.at[rbuf1], sd1).wait()
    plsc.subcore_barrier()
    pltpu.sync_copy(deg_sp.at[pl.ds(s * RPT, RPT)],
                    out_ref.at[c, pl.ds(s * RPT, RPT)])


def _degrees(rows2):
    mesh = plsc.VectorSubcoreMesh(core_axis_name="c", subcore_axis_name="s")
    return pl.kernel(
        _deg_body,
        out_type=jax.ShapeDtypeStruct((NC, NP, REC), jnp.float32),
        mesh=mesh,
        scratch_types=[
            pltpu.VMEM((BB, REC), jnp.float32),
            pltpu.VMEM((BB,), jnp.int32),
            pltpu.VMEM((BB,), jnp.int32),
            pltpu.SemaphoreType.DMA,
            pltpu.SemaphoreType.DMA,
            pltpu.VMEM_SHARED((NP, REC), jnp.float32),
        ],
    )(rows2)


def _scale_body(xh_ref, degp_ref, y_ref):
    xh = xh_ref[...]
    for c in range(NC):
        deg = degp_ref[c][:, 0:1]
        dinv = jnp.where(deg > 0, lax.rsqrt(deg), 0.0)
        y_ref[pl.ds(c * NP, NP), :] = xh * dinv


def _scale(xh, degp):
    return pl.pallas_call(
        _scale_body,
        out_shape=jax.ShapeDtypeStruct((NC * NP, REC), jnp.float32),
    )(xh, degp)


def _agg_body(y_ref, idx_ref, bh1_ref, out_ref,
              ybuf0, ybuf1, bbuf, ibuf0, ibuf1, sg0, sg1, ss0, ss1,
              a_sp, bh_sp):
    c = lax.axis_index("c")
    s = lax.axis_index("s")
    zero16 = jnp.zeros((16,), jnp.float32)
    lane0 = lax.iota(jnp.int32, 16) == 0

    def zrow(j, carry):
        for k in range(REC // 16):
            ybuf0[j, pl.ds(16 * k, 16)] = zero16
        return carry

    lax.fori_loop(0, BA, zrow, 0)
    for q in range(RPT // BA):
        pltpu.sync_copy(ybuf0, a_sp.at[pl.ds(s * RPT + q * BA, BA)])

    @pl.when(s == 0)
    def _():
        pltpu.sync_copy(bh1_ref, bh_sp)

    plsc.subcore_barrier()

    def make_edge(ybuf, off):
        def edge(j, carry):
            for k in range(4):
                ul = ybuf[off + j, pl.ds(16 * k, 16)]
                uh = ybuf[off + j, pl.ds(64 + 16 * k, 16)]
                vl = bbuf[j, pl.ds(16 * k, 16)]
                vh = bbuf[j, pl.ds(64 + 16 * k, 16)]
                if k == 0:
                    v2l = jnp.where(lane0, 0.0, vh)
                    v2h = jnp.where(lane0, vh, vl)
                else:
                    v2l, v2h = vh, vl
                ybuf[off + j, pl.ds(16 * k, 16)] = ul * vl + uh * vh
                ybuf[off + j, pl.ds(64 + 16 * k, 16)] = ul * v2l - uh * v2h
            return carry
        return edge

    base = s * EPT

    def phase(b, ybuf, ibuf, sem, ibuf_n, ybuf_n, sem_n, ss, ss_n):
        @pl.when(b >= 1)
        def _():
            pltpu.make_async_copy(ybuf_n, a_sp.at[ibuf_n.at[0]], ss_n).wait()

        @pl.when(b + 1 < NBLKA)
        def _():
            pltpu.sync_copy(idx_ref.at[c, :, pl.ds(base + (b + 1) * BA, BA)],
                            ibuf_n)
            pltpu.async_copy(y_ref.at[ibuf_n.at[1]], ybuf_n, sem_n)

        pltpu.make_async_copy(y_ref.at[ibuf.at[1]], ybuf, sem).wait()
        for h in range(BA // BH):
            pltpu.sync_copy(bh_sp.at[ibuf.at[2, pl.ds(BH * h, BH)]], bbuf)
            lax.fori_loop(0, BH, make_edge(ybuf, BH * h), 0)
        pltpu.async_copy(ybuf, a_sp.at[ibuf.at[0]], ss, add=True)

    pltpu.sync_copy(idx_ref.at[c, :, pl.ds(base, BA)], ibuf0)
    pltpu.async_copy(y_ref.at[ibuf0.at[1]], ybuf0, sg0)

    def blk2(b2, carry):
        phase(2 * b2, ybuf0, ibuf0, sg0, ibuf1, ybuf1, sg1, ss0, ss1)
        phase(2 * b2 + 1, ybuf1, ibuf1, sg1, ibuf0, ybuf0, sg0, ss1, ss0)
        return carry

    lax.fori_loop(0, NBLKA // 2, blk2, 0)
    pltpu.make_async_copy(ybuf1, a_sp.at[ibuf1.at[0]], ss1).wait()
    plsc.subcore_barrier()
    pltpu.sync_copy(a_sp.at[pl.ds(s * RPT, RPT)],
                    out_ref.at[c, pl.ds(s * RPT, RPT)])


def _aggregate(yflat, idx3, bh1):
    mesh = plsc.VectorSubcoreMesh(core_axis_name="c", subcore_axis_name="s")
    return pl.kernel(
        _agg_body,
        out_type=jax.ShapeDtypeStruct((NC, NP, REC), jnp.float32),
        mesh=mesh,
        scratch_types=[
            pltpu.VMEM((BA, REC), jnp.float32),
            pltpu.VMEM((BA, REC), jnp.float32),
            pltpu.VMEM((BH, REC), jnp.float32),
            pltpu.VMEM((3, BA), jnp.int32),
            pltpu.VMEM((3, BA), jnp.int32),
            pltpu.SemaphoreType.DMA,
            pltpu.SemaphoreType.DMA,
            pltpu.SemaphoreType.DMA,
            pltpu.SemaphoreType.DMA,
            pltpu.VMEM_SHARED((NP, REC), jnp.float32),
            pltpu.VMEM_SHARED((RELP, REC), jnp.float32),
        ],
    )(yflat, idx3, bh1)


GB = 8
NBR = NP // GB


def _combine_body(ain_ref, aout_ref, xh_ref, ldual_ref, w2in_ref, w2out_ref,
                  w2loop_ref, degin_ref, degout_ref, bias_ref,
                  msg_ref, stats_ref):
    i = pl.program_id(0)

    def dinv(deg_ref):
        deg = deg_ref[0][:, 0:1]
        return jnp.where(deg > 0, lax.rsqrt(deg), 0.0)

    m_in = _dot(ain_ref[0], w2in_ref[...]) * dinv(degin_ref)
    m_out = _dot(aout_ref[0], w2out_ref[...]) * dinv(degout_ref)

    lrec = ldual_ref[0:1, :]
    sl = _cplx_dense(xh_ref[...], lrec[:, :REC], lrec[:, REC:])
    m_loop = _dot(sl, w2loop_ref[...])

    msg = (m_in + m_out + m_loop) * (1.0 / 3.0) + bias_ref[...][None, :]
    msg_ref[...] = msg

    rowid = i * NBR + lax.broadcasted_iota(jnp.int32, (NBR, D), 0)
    msgv = jnp.where(rowid < N, msg, 0.0)
    s1 = jnp.sum(msgv, axis=0, keepdims=True)
    s2 = jnp.sum(msgv * msgv, axis=0, keepdims=True)
    stats_ref[...] = jnp.concatenate(
        [s1, s2, jnp.zeros((6, D), jnp.float32)], axis=0)[None]


def _combine(a2, xh, ldual, w2in, w2out, w2loop, degp, bias):
    full = lambda *dims: pl.BlockSpec(dims, lambda i: (0,) * len(dims))
    return pl.pallas_call(
        _combine_body,
        grid=(GB,),
        in_specs=[
            pl.BlockSpec((1, NBR, REC), lambda i: (0, i, 0)),
            pl.BlockSpec((1, NBR, REC), lambda i: (1, i, 0)),
            pl.BlockSpec((NBR, REC), lambda i: (i, 0)),
            full(8, VREC),
            full(REC, D),
            full(REC, D),
            full(REC, D),
            pl.BlockSpec((1, NBR, REC), lambda i: (0, i, 0)),
            pl.BlockSpec((1, NBR, REC), lambda i: (1, i, 0)),
            full(D),
        ],
        out_specs=[
            pl.BlockSpec((NBR, D), lambda i: (i, 0)),
            pl.BlockSpec((1, 8, D), lambda i: (i, 0, 0)),
        ],
        out_shape=[
            jax.ShapeDtypeStruct((NP, D), jnp.float32),
            jax.ShapeDtypeStruct((GB, 8, D), jnp.float32),
        ],
    )(a2, a2, xh, ldual, w2in, w2out, w2loop, degp, degp, bias)


def _bn_body(msg_ref, stats_ref, gamma_ref, beta_ref, out_ref):
    mean = jnp.sum(stats_ref[:, 0, :], axis=0, keepdims=True) * (1.0 / N)
    ex2 = jnp.sum(stats_ref[:, 1, :], axis=0, keepdims=True) * (1.0 / N)
    var = ex2 - mean * mean
    inv = lax.rsqrt(var + 1e-5)
    out_ref[...] = ((msg_ref[...] - mean) * inv * gamma_ref[...][None, :]
                    + beta_ref[...][None, :])


GB2 = 10
NBR2 = N // GB2


def _bn(msg, stats, gamma, beta):
    return pl.pallas_call(
        _bn_body,
        grid=(GB2,),
        in_specs=[
            pl.BlockSpec((NBR2, D), lambda i: (i, 0)),
            pl.BlockSpec((GB, 8, D), lambda i: (0, 0, 0)),
            pl.BlockSpec((D,), lambda i: (0,)),
            pl.BlockSpec((D,), lambda i: (0,)),
        ],
        out_specs=pl.BlockSpec((NBR2, D), lambda i: (i, 0)),
        out_shape=jax.ShapeDtypeStruct((N, D), jnp.float32),
    )(msg, stats, gamma, beta)


def kernel(x, edge_index, edge_type, rel_embed, w_loop, w_in, w_out, w_rel,
           loop_rel, bias, bn_gamma, bn_beta):
    fmat = jnp.asarray(_F2)
    fdmat = jnp.asarray(_FD)
    pmat = jnp.asarray(_P2)

    rows2, idx3 = _packidx(edge_index, edge_type)
    xh, bh1, ldual, w2in, w2out, w2loop, relout = _prep(
        x, rel_embed, loop_rel, w_in, w_out, w_loop, w_rel,
        fmat, fdmat, pmat)
    degp = _degrees(rows2)
    yflat = _scale(xh, degp)
    a2 = _aggregate(yflat, idx3, bh1)
    msg, stats = _combine(a2, xh, ldual, w2in, w2out, w2loop, degp, bias)
    out = _bn(msg, stats, bn_gamma, bn_beta)
    return out, relout
